# Initial kernel scaffold; baseline (speedup 1.0000x reference)
import jax
import jax.numpy as jnp
from jax.experimental import pallas as pl


def kernel(V, edge_attr, edge_index, rev_index, batch_ids, W_i, b_i, W_h, W_o, b_o, bn_gamma, bn_beta, bn_mean, bn_var, W_ffn, b_ffn):
    raise NotImplementedError("write your pallas kernel here")


# speedup vs baseline: 2.4116x; 2.4116x over previous
import functools

import jax
import jax.numpy as jnp
from jax import lax
from jax.experimental import pallas as pl
from jax.experimental.pallas import tpu as pltpu
from jax.experimental.pallas import tpu_sc as plsc

N = 10000
EH = 80000
E = 2 * EH
DV = 256
DE = 16
HID = 256
B = 64
OUT = 300

EBLK = 1000
NEB = E // EBLK
NBLK = 1000
NNB = N // NBLK

SC_CORES = 2
SC_SUBCORES = 16
CHUNK = 256
GCHUNK = 128
ROWBLK = 200
HALF = HID // 2


def _mm2_body(v_ref, wi1_ref, wo1_ref, bo_ref, vwi_ref, vwo_ref):
    v = v_ref[...]
    vwi_ref[...] = jnp.dot(v, wi1_ref[...], preferred_element_type=jnp.float32)
    vwo_ref[...] = jnp.dot(v, wo1_ref[...], preferred_element_type=jnp.float32) + bo_ref[...]


def _node_matmuls(V, W_i1, W_o1, b_o):
    return pl.pallas_call(
        _mm2_body,
        grid=(NNB,),
        in_specs=[
            pl.BlockSpec((NBLK, DV), lambda i: (i, 0)),
            pl.BlockSpec((DV, HID), lambda i: (0, 0)),
            pl.BlockSpec((DV, HID), lambda i: (0, 0)),
            pl.BlockSpec((1, HID), lambda i: (0, 0)),
        ],
        out_specs=[
            pl.BlockSpec((NBLK, HID), lambda i: (i, 0)),
            pl.BlockSpec((NBLK, HID), lambda i: (i, 0)),
        ],
        out_shape=[
            jax.ShapeDtypeStruct((N, HID), jnp.float32),
            jax.ShapeDtypeStruct((N, HID), jnp.float32),
        ],
    )(V, W_i1, W_o1, b_o)


def _h0_hw_body(g_ref, ea_ref, wi2_ref, bi_ref, wh_ref, h_ref, hw_ref):
    x = g_ref[...] + jnp.dot(ea_ref[...], wi2_ref[...],
                             preferred_element_type=jnp.float32) + bi_ref[...]
    h = jnp.maximum(x, 0.0)
    h_ref[...] = h
    hw_ref[...] = jnp.dot(h, wh_ref[...], preferred_element_type=jnp.float32)


def _h0_and_hw(G0, edge_attr, W_i2, b_i, W_h):
    return pl.pallas_call(
        _h0_hw_body,
        grid=(NEB,),
        in_specs=[
            pl.BlockSpec((EBLK, HID), lambda i: (i, 0)),
            pl.BlockSpec((EBLK, DE), lambda i: (i, 0)),
            pl.BlockSpec((DE, HID), lambda i: (0, 0)),
            pl.BlockSpec((1, HID), lambda i: (0, 0)),
            pl.BlockSpec((HID, HID), lambda i: (0, 0)),
        ],
        out_specs=[
            pl.BlockSpec((EBLK, HID), lambda i: (i, 0)),
            pl.BlockSpec((EBLK, HID), lambda i: (i, 0)),
        ],
        out_shape=[
            jax.ShapeDtypeStruct((E, HID), jnp.float32),
            jax.ShapeDtypeStruct((E, HID), jnp.float32),
        ],
    )(G0, edge_attr, W_i2, b_i, W_h)


def _step_hw_body(h0_ref, g_ref, hwp_ref, wh_ref, hw_ref):
    h = jnp.maximum(h0_ref[...] + g_ref[...] - hwp_ref[...], 0.0)
    hw_ref[...] = jnp.dot(h, wh_ref[...], preferred_element_type=jnp.float32)


def _step_hw(h0, G, hWprev, W_h):
    return pl.pallas_call(
        _step_hw_body,
        grid=(NEB,),
        in_specs=[
            pl.BlockSpec((EBLK, HID), lambda i: (i, 0)),
            pl.BlockSpec((EBLK, HID), lambda i: (i, 0)),
            pl.BlockSpec((EBLK, HID), lambda i: ((i + NEB // 2) % NEB, 0)),
            pl.BlockSpec((HID, HID), lambda i: (0, 0)),
        ],
        out_specs=pl.BlockSpec((EBLK, HID), lambda i: (i, 0)),
        out_shape=jax.ShapeDtypeStruct((E, HID), jnp.float32),
    )(h0, G, hWprev, W_h)


def _step_h_body(h0_ref, g_ref, hwp_ref, h_ref):
    h_ref[...] = jnp.maximum(h0_ref[...] + g_ref[...] - hwp_ref[...], 0.0)


def _step_h(h0, G, hWprev):
    return pl.pallas_call(
        _step_h_body,
        grid=(NEB,),
        in_specs=[
            pl.BlockSpec((EBLK, HID), lambda i: (i, 0)),
            pl.BlockSpec((EBLK, HID), lambda i: (i, 0)),
            pl.BlockSpec((EBLK, HID), lambda i: ((i + NEB // 2) % NEB, 0)),
        ],
        out_specs=pl.BlockSpec((EBLK, HID), lambda i: (i, 0)),
        out_shape=jax.ShapeDtypeStruct((E, HID), jnp.float32),
    )(h0, G, hWprev)


def _final_body(vwo_ref, mv_ref, wo2_ref, ids_ref, gam_ref, bet_ref,
                mean_ref, var_ref, wf_ref, bf_ref, out_ref,
                sum_ref, cnt_ref):
    i = pl.program_id(0)
    hv = jnp.maximum(
        vwo_ref[...] + jnp.dot(mv_ref[...], wo2_ref[...],
                               preferred_element_type=jnp.float32), 0.0)
    ids = ids_ref[0, 0, :]
    oneh = (lax.broadcasted_iota(jnp.int32, (NBLK, B), 1)
            == ids[:, None]).astype(jnp.float32)
    psum = lax.dot_general(oneh, hv, (((0,), (0,)), ((), ())),
                           preferred_element_type=jnp.float32)
    pcnt = lax.dot_general(oneh, jnp.ones((NBLK, 128), jnp.float32),
                           (((0,), (0,)), ((), ())),
                           preferred_element_type=jnp.float32)

    @pl.when(i == 0)
    def _():
        sum_ref[...] = jnp.zeros_like(sum_ref)
        cnt_ref[...] = jnp.zeros_like(cnt_ref)

    sum_ref[...] += psum
    cnt_ref[...] += pcnt

    @pl.when(i == NNB - 1)
    def _():
        counts = jnp.maximum(cnt_ref[:, 0:1], 1.0)
        hm = sum_ref[...] / counts
        hn = (gam_ref[...] * (hm - mean_ref[...])
              * lax.rsqrt(var_ref[...] + 1e-5) + bet_ref[...])
        out_ref[...] = jnp.maximum(
            jnp.dot(hn, wf_ref[...], preferred_element_type=jnp.float32)
            + bf_ref[...], 0.0)


def _final(VWo, m_v, W_o2, ids3, bn_gamma, bn_beta, bn_mean, bn_var,
           W_ffn, b_ffn):
    return pl.pallas_call(
        _final_body,
        grid=(NNB,),
        in_specs=[
            pl.BlockSpec((NBLK, HID), lambda i: (i, 0)),
            pl.BlockSpec((NBLK, HID), lambda i: (i, 0)),
            pl.BlockSpec((HID, HID), lambda i: (0, 0)),
            pl.BlockSpec((1, 1, NBLK), lambda i: (i, 0, 0)),
            pl.BlockSpec((1, HID), lambda i: (0, 0)),
            pl.BlockSpec((1, HID), lambda i: (0, 0)),
            pl.BlockSpec((1, HID), lambda i: (0, 0)),
            pl.BlockSpec((1, HID), lambda i: (0, 0)),
            pl.BlockSpec((HID, OUT), lambda i: (0, 0)),
            pl.BlockSpec((1, OUT), lambda i: (0, 0)),
        ],
        out_specs=pl.BlockSpec((B, OUT), lambda i: (0, 0)),
        out_shape=jax.ShapeDtypeStruct((B, OUT), jnp.float32),
        scratch_shapes=[
            pltpu.VMEM((B, HID), jnp.float32),
            pltpu.VMEM((B, 128), jnp.float32),
        ],
    )(VWo, m_v, W_o2, ids3, bn_gamma, bn_beta, bn_mean, bn_var, W_ffn, b_ffn)


def _sc_mesh():
    return plsc.VectorSubcoreMesh(core_axis_name="c", subcore_axis_name="s")


def _gather_rows(table, src2):
    nch = E // GCHUNK
    nw = SC_CORES * SC_SUBCORES
    nloop = -(-nch // nw)

    @functools.partial(
        pl.kernel,
        mesh=_sc_mesh(),
        out_type=jax.ShapeDtypeStruct((E, HID), jnp.float32),
        scratch_types=[
            pltpu.VMEM((GCHUNK,), jnp.int32),
            pltpu.VMEM((GCHUNK, HID), jnp.float32),
        ],
    )
    def k(table_hbm, src_hbm, out_hbm, idx_v, rows_v):
        c = lax.axis_index("c")
        s = lax.axis_index("s")
        wid = s * SC_CORES + c

        @pl.loop(0, nloop)
        def _(j):
            ci = j * nw + wid

            @pl.when(ci < nch)
            def _():
                base = ci * GCHUNK
                pltpu.sync_copy(src_hbm.at[ci], idx_v)
                pltpu.sync_copy(table_hbm.at[idx_v], rows_v)
                pltpu.sync_copy(rows_v, out_hbm.at[pl.ds(base, GCHUNK)])

    return k(table, src2)


def _segsum_gather(x, dst2, src2, zrows):
    nch = E // CHUNK
    nloop = -(-nch // SC_SUBCORES)

    @functools.partial(
        pl.kernel,
        mesh=_sc_mesh(),
        out_type=jax.ShapeDtypeStruct((E, HID), jnp.float32),
        scratch_types=[
            pltpu.VMEM_SHARED((N, HALF), jnp.float32),
            pltpu.VMEM((CHUNK // 128, 128), jnp.int32),
            pltpu.VMEM((CHUNK, HALF), jnp.float32),
        ],
    )
    def k(x_hbm, dst_hbm, src_hbm, z_hbm, g_hbm, acc_sh, idx_v, buf_v):
        c = lax.axis_index("c")
        s = lax.axis_index("s")
        col0 = c * HALF

        @pl.loop(0, -(-(N // ROWBLK) // SC_SUBCORES))
        def _(j):
            rb = j * SC_SUBCORES + s

            @pl.when(rb < N // ROWBLK)
            def _():
                pltpu.sync_copy(z_hbm, acc_sh.at[pl.ds(rb * ROWBLK, ROWBLK)])

        plsc.subcore_barrier()

        @pl.loop(0, nloop)
        def _(j):
            ci = j * SC_SUBCORES + s

            @pl.when(ci < nch)
            def _():
                base = ci * CHUNK
                pltpu.sync_copy(dst_hbm.at[pl.ds(ci * (CHUNK // 128), CHUNK // 128)], idx_v)
                pltpu.sync_copy(x_hbm.at[pl.ds(base, CHUNK), pl.ds(col0, HALF)], buf_v)
                for q in range(CHUNK // 128):
                    pltpu.sync_copy(buf_v.at[pl.ds(q * 128, 128)],
                                    acc_sh.at[idx_v.at[q]], add=True)

        plsc.subcore_barrier()

        @pl.loop(0, nloop)
        def _(j):
            ci = j * SC_SUBCORES + s

            @pl.when(ci < nch)
            def _():
                base = ci * CHUNK
                pltpu.sync_copy(src_hbm.at[pl.ds(ci * (CHUNK // 128), CHUNK // 128)], idx_v)
                for q in range(CHUNK // 128):
                    pltpu.sync_copy(acc_sh.at[idx_v.at[q]],
                                    buf_v.at[pl.ds(q * 128, 128)])
                pltpu.sync_copy(buf_v, g_hbm.at[pl.ds(base, CHUNK), pl.ds(col0, HALF)])

    return k(x, dst2, src2, zrows)


def _segsum(x, dst2, zrows):
    nch = E // CHUNK
    nloop = -(-nch // SC_SUBCORES)

    @functools.partial(
        pl.kernel,
        mesh=_sc_mesh(),
        out_type=jax.ShapeDtypeStruct((N, HID), jnp.float32),
        scratch_types=[
            pltpu.VMEM_SHARED((N, HALF), jnp.float32),
            pltpu.VMEM((CHUNK // 128, 128), jnp.int32),
            pltpu.VMEM((CHUNK, HALF), jnp.float32),
        ],
    )
    def k(x_hbm, dst_hbm, z_hbm, s_hbm, acc_sh, idx_v, buf_v):
        c = lax.axis_index("c")
        s = lax.axis_index("s")
        col0 = c * HALF

        @pl.loop(0, -(-(N // ROWBLK) // SC_SUBCORES))
        def _(j):
            rb = j * SC_SUBCORES + s

            @pl.when(rb < N // ROWBLK)
            def _():
                pltpu.sync_copy(z_hbm, acc_sh.at[pl.ds(rb * ROWBLK, ROWBLK)])

        plsc.subcore_barrier()

        @pl.loop(0, nloop)
        def _(j):
            ci = j * SC_SUBCORES + s

            @pl.when(ci < nch)
            def _():
                base = ci * CHUNK
                pltpu.sync_copy(dst_hbm.at[pl.ds(ci * (CHUNK // 128), CHUNK // 128)], idx_v)
                pltpu.sync_copy(x_hbm.at[pl.ds(base, CHUNK), pl.ds(col0, HALF)], buf_v)
                for q in range(CHUNK // 128):
                    pltpu.sync_copy(buf_v.at[pl.ds(q * 128, 128)],
                                    acc_sh.at[idx_v.at[q]], add=True)

        plsc.subcore_barrier()

        @pl.loop(0, -(-(N // ROWBLK) // SC_SUBCORES))
        def _(j):
            rb = j * SC_SUBCORES + s

            @pl.when(rb < N // ROWBLK)
            def _():
                pltpu.sync_copy(acc_sh.at[pl.ds(rb * ROWBLK, ROWBLK)],
                                s_hbm.at[pl.ds(rb * ROWBLK, ROWBLK),
                                         pl.ds(col0, HALF)])

    return k(x, dst2, zrows)


def kernel(V, edge_attr, edge_index, rev_index, batch_ids,
           W_i, b_i, W_h, W_o, b_o, bn_gamma, bn_beta, bn_mean, bn_var,
           W_ffn, b_ffn):
    src = edge_index[0]
    dst = edge_index[1]
    W_i1 = W_i[:DV]
    W_i2 = W_i[DV:]
    W_o1 = W_o[:DV]
    W_o2 = W_o[DV:]
    b_i2 = b_i.reshape(1, HID)
    b_o2 = b_o.reshape(1, HID)
    zrows = jnp.zeros((ROWBLK, HALF), jnp.float32)
    ids3 = batch_ids.reshape(NNB, 1, NBLK)
    src2 = src.reshape(E // 128, 128)
    dst2 = dst.reshape(E // 128, 128)

    VWi, VWo = _node_matmuls(V, W_i1, W_o1, b_o2)
    G0 = _gather_rows(VWi, src2)
    h0, hW = _h0_and_hw(G0, edge_attr, W_i2, b_i2, W_h)
    G = _segsum_gather(hW, dst2, src2, zrows)
    hW = _step_hw(h0, G, hW, W_h)
    G = _segsum_gather(hW, dst2, src2, zrows)
    h3 = _step_h(h0, G, hW)
    m_v = _segsum(h3, dst2, zrows)
    return _final(VWo, m_v, W_o2, ids3, bn_gamma.reshape(1, HID),
                  bn_beta.reshape(1, HID), bn_mean.reshape(1, HID),
                  bn_var.reshape(1, HID), W_ffn, b_ffn.reshape(1, OUT))


# speedup vs baseline: 2.4140x; 1.0010x over previous
import functools

import jax
import jax.numpy as jnp
from jax import lax
from jax.experimental import pallas as pl
from jax.experimental.pallas import tpu as pltpu
from jax.experimental.pallas import tpu_sc as plsc

N = 10000
EH = 80000
E = 2 * EH
DV = 256
DE = 16
HID = 256
B = 64
OUT = 300

EBLK = 1000
NEB = E // EBLK
NBLK = 1000
NNB = N // NBLK

SC_CORES = 2
SC_SUBCORES = 16
CHUNK = 256
GCHUNK = 128
ROWBLK = 200
HALF = HID // 2


def _mm2_body(v_ref, wi1_ref, wo1_ref, bo_ref, vwi_ref, vwo_ref):
    v = v_ref[...]
    vwi_ref[...] = jnp.dot(v, wi1_ref[...], preferred_element_type=jnp.float32)
    vwo_ref[...] = jnp.dot(v, wo1_ref[...], preferred_element_type=jnp.float32) + bo_ref[...]


def _node_matmuls(V, W_i1, W_o1, b_o):
    return pl.pallas_call(
        _mm2_body,
        grid=(NNB,),
        in_specs=[
            pl.BlockSpec((NBLK, DV), lambda i: (i, 0)),
            pl.BlockSpec((DV, HID), lambda i: (0, 0)),
            pl.BlockSpec((DV, HID), lambda i: (0, 0)),
            pl.BlockSpec((1, HID), lambda i: (0, 0)),
        ],
        out_specs=[
            pl.BlockSpec((NBLK, HID), lambda i: (i, 0)),
            pl.BlockSpec((NBLK, HID), lambda i: (i, 0)),
        ],
        out_shape=[
            jax.ShapeDtypeStruct((N, HID), jnp.float32),
            jax.ShapeDtypeStruct((N, HID), jnp.float32),
        ],
        compiler_params=pltpu.CompilerParams(dimension_semantics=("parallel",)),
    )(V, W_i1, W_o1, b_o)


def _h0_hw_body(g_ref, ea_ref, wi2_ref, bi_ref, wh_ref, h_ref, hw_ref):
    x = g_ref[...] + jnp.dot(ea_ref[...], wi2_ref[...],
                             preferred_element_type=jnp.float32) + bi_ref[...]
    h = jnp.maximum(x, 0.0)
    h_ref[...] = h
    hw_ref[...] = jnp.dot(h, wh_ref[...], preferred_element_type=jnp.float32)


def _h0_and_hw(G0, edge_attr, W_i2, b_i, W_h):
    return pl.pallas_call(
        _h0_hw_body,
        grid=(NEB,),
        in_specs=[
            pl.BlockSpec((EBLK, HID), lambda i: (i, 0)),
            pl.BlockSpec((EBLK, DE), lambda i: (i, 0)),
            pl.BlockSpec((DE, HID), lambda i: (0, 0)),
            pl.BlockSpec((1, HID), lambda i: (0, 0)),
            pl.BlockSpec((HID, HID), lambda i: (0, 0)),
        ],
        out_specs=[
            pl.BlockSpec((EBLK, HID), lambda i: (i, 0)),
            pl.BlockSpec((EBLK, HID), lambda i: (i, 0)),
        ],
        out_shape=[
            jax.ShapeDtypeStruct((E, HID), jnp.float32),
            jax.ShapeDtypeStruct((E, HID), jnp.float32),
        ],
        compiler_params=pltpu.CompilerParams(dimension_semantics=("parallel",)),
    )(G0, edge_attr, W_i2, b_i, W_h)


def _step_hw_body(h0_ref, g_ref, hwp_ref, wh_ref, hw_ref):
    h = jnp.maximum(h0_ref[...] + g_ref[...] - hwp_ref[...], 0.0)
    hw_ref[...] = jnp.dot(h, wh_ref[...], preferred_element_type=jnp.float32)


def _step_hw(h0, G, hWprev, W_h):
    return pl.pallas_call(
        _step_hw_body,
        grid=(NEB,),
        in_specs=[
            pl.BlockSpec((EBLK, HID), lambda i: (i, 0)),
            pl.BlockSpec((EBLK, HID), lambda i: (i, 0)),
            pl.BlockSpec((EBLK, HID), lambda i: ((i + NEB // 2) % NEB, 0)),
            pl.BlockSpec((HID, HID), lambda i: (0, 0)),
        ],
        out_specs=pl.BlockSpec((EBLK, HID), lambda i: (i, 0)),
        out_shape=jax.ShapeDtypeStruct((E, HID), jnp.float32),
        compiler_params=pltpu.CompilerParams(dimension_semantics=("parallel",)),
    )(h0, G, hWprev, W_h)


def _step_h_body(h0_ref, g_ref, hwp_ref, h_ref):
    h_ref[...] = jnp.maximum(h0_ref[...] + g_ref[...] - hwp_ref[...], 0.0)


def _step_h(h0, G, hWprev):
    return pl.pallas_call(
        _step_h_body,
        grid=(NEB,),
        in_specs=[
            pl.BlockSpec((EBLK, HID), lambda i: (i, 0)),
            pl.BlockSpec((EBLK, HID), lambda i: (i, 0)),
            pl.BlockSpec((EBLK, HID), lambda i: ((i + NEB // 2) % NEB, 0)),
        ],
        out_specs=pl.BlockSpec((EBLK, HID), lambda i: (i, 0)),
        out_shape=jax.ShapeDtypeStruct((E, HID), jnp.float32),
        compiler_params=pltpu.CompilerParams(dimension_semantics=("parallel",)),
    )(h0, G, hWprev)


def _final_body(vwo_ref, mv_ref, wo2_ref, ids_ref, gam_ref, bet_ref,
                mean_ref, var_ref, wf_ref, bf_ref, out_ref,
                sum_ref, cnt_ref):
    i = pl.program_id(0)
    hv = jnp.maximum(
        vwo_ref[...] + jnp.dot(mv_ref[...], wo2_ref[...],
                               preferred_element_type=jnp.float32), 0.0)
    ids = ids_ref[0, 0, :]
    oneh = (lax.broadcasted_iota(jnp.int32, (NBLK, B), 1)
            == ids[:, None]).astype(jnp.float32)
    psum = lax.dot_general(oneh, hv, (((0,), (0,)), ((), ())),
                           preferred_element_type=jnp.float32)
    pcnt = lax.dot_general(oneh, jnp.ones((NBLK, 128), jnp.float32),
                           (((0,), (0,)), ((), ())),
                           preferred_element_type=jnp.float32)

    @pl.when(i == 0)
    def _():
        sum_ref[...] = jnp.zeros_like(sum_ref)
        cnt_ref[...] = jnp.zeros_like(cnt_ref)

    sum_ref[...] += psum
    cnt_ref[...] += pcnt

    @pl.when(i == NNB - 1)
    def _():
        counts = jnp.maximum(cnt_ref[:, 0:1], 1.0)
        hm = sum_ref[...] / counts
        hn = (gam_ref[...] * (hm - mean_ref[...])
              * lax.rsqrt(var_ref[...] + 1e-5) + bet_ref[...])
        out_ref[...] = jnp.maximum(
            jnp.dot(hn, wf_ref[...], preferred_element_type=jnp.float32)
            + bf_ref[...], 0.0)


def _final(VWo, m_v, W_o2, ids3, bn_gamma, bn_beta, bn_mean, bn_var,
           W_ffn, b_ffn):
    return pl.pallas_call(
        _final_body,
        grid=(NNB,),
        in_specs=[
            pl.BlockSpec((NBLK, HID), lambda i: (i, 0)),
            pl.BlockSpec((NBLK, HID), lambda i: (i, 0)),
            pl.BlockSpec((HID, HID), lambda i: (0, 0)),
            pl.BlockSpec((1, 1, NBLK), lambda i: (i, 0, 0)),
            pl.BlockSpec((1, HID), lambda i: (0, 0)),
            pl.BlockSpec((1, HID), lambda i: (0, 0)),
            pl.BlockSpec((1, HID), lambda i: (0, 0)),
            pl.BlockSpec((1, HID), lambda i: (0, 0)),
            pl.BlockSpec((HID, OUT), lambda i: (0, 0)),
            pl.BlockSpec((1, OUT), lambda i: (0, 0)),
        ],
        out_specs=pl.BlockSpec((B, OUT), lambda i: (0, 0)),
        out_shape=jax.ShapeDtypeStruct((B, OUT), jnp.float32),
        scratch_shapes=[
            pltpu.VMEM((B, HID), jnp.float32),
            pltpu.VMEM((B, 128), jnp.float32),
        ],
    )(VWo, m_v, W_o2, ids3, bn_gamma, bn_beta, bn_mean, bn_var, W_ffn, b_ffn)


def _sc_mesh():
    return plsc.VectorSubcoreMesh(core_axis_name="c", subcore_axis_name="s")


def _gather_rows(table, src2):
    nch = E // GCHUNK
    nw = SC_CORES * SC_SUBCORES
    nloop = -(-nch // nw)

    @functools.partial(
        pl.kernel,
        mesh=_sc_mesh(),
        out_type=jax.ShapeDtypeStruct((E, HID), jnp.float32),
        scratch_types=[
            pltpu.VMEM((GCHUNK,), jnp.int32),
            pltpu.VMEM((GCHUNK, HID), jnp.float32),
        ],
    )
    def k(table_hbm, src_hbm, out_hbm, idx_v, rows_v):
        c = lax.axis_index("c")
        s = lax.axis_index("s")
        wid = s * SC_CORES + c

        @pl.loop(0, nloop)
        def _(j):
            ci = j * nw + wid

            @pl.when(ci < nch)
            def _():
                base = ci * GCHUNK
                pltpu.sync_copy(src_hbm.at[ci], idx_v)
                pltpu.sync_copy(table_hbm.at[idx_v], rows_v)
                pltpu.sync_copy(rows_v, out_hbm.at[pl.ds(base, GCHUNK)])

    return k(table, src2)


def _segsum_gather(x, dst2, src2, zrows):
    nch = E // CHUNK
    nloop = -(-nch // SC_SUBCORES)

    @functools.partial(
        pl.kernel,
        mesh=_sc_mesh(),
        out_type=jax.ShapeDtypeStruct((E, HID), jnp.float32),
        scratch_types=[
            pltpu.VMEM_SHARED((N, HALF), jnp.float32),
            pltpu.VMEM((CHUNK // 128, 128), jnp.int32),
            pltpu.VMEM((CHUNK, HALF), jnp.float32),
        ],
    )
    def k(x_hbm, dst_hbm, src_hbm, z_hbm, g_hbm, acc_sh, idx_v, buf_v):
        c = lax.axis_index("c")
        s = lax.axis_index("s")
        col0 = c * HALF

        @pl.loop(0, -(-(N // ROWBLK) // SC_SUBCORES))
        def _(j):
            rb = j * SC_SUBCORES + s

            @pl.when(rb < N // ROWBLK)
            def _():
                pltpu.sync_copy(z_hbm, acc_sh.at[pl.ds(rb * ROWBLK, ROWBLK)])

        plsc.subcore_barrier()

        @pl.loop(0, nloop)
        def _(j):
            ci = j * SC_SUBCORES + s

            @pl.when(ci < nch)
            def _():
                base = ci * CHUNK
                pltpu.sync_copy(dst_hbm.at[pl.ds(ci * (CHUNK // 128), CHUNK // 128)], idx_v)
                pltpu.sync_copy(x_hbm.at[pl.ds(base, CHUNK), pl.ds(col0, HALF)], buf_v)
                for q in range(CHUNK // 128):
                    pltpu.sync_copy(buf_v.at[pl.ds(q * 128, 128)],
                                    acc_sh.at[idx_v.at[q]], add=True)

        plsc.subcore_barrier()

        @pl.loop(0, nloop)
        def _(j):
            ci = j * SC_SUBCORES + s

            @pl.when(ci < nch)
            def _():
                base = ci * CHUNK
                pltpu.sync_copy(src_hbm.at[pl.ds(ci * (CHUNK // 128), CHUNK // 128)], idx_v)
                for q in range(CHUNK // 128):
                    pltpu.sync_copy(acc_sh.at[idx_v.at[q]],
                                    buf_v.at[pl.ds(q * 128, 128)])
                pltpu.sync_copy(buf_v, g_hbm.at[pl.ds(base, CHUNK), pl.ds(col0, HALF)])

    return k(x, dst2, src2, zrows)


def _segsum(x, dst2, zrows):
    nch = E // CHUNK
    nloop = -(-nch // SC_SUBCORES)

    @functools.partial(
        pl.kernel,
        mesh=_sc_mesh(),
        out_type=jax.ShapeDtypeStruct((N, HID), jnp.float32),
        scratch_types=[
            pltpu.VMEM_SHARED((N, HALF), jnp.float32),
            pltpu.VMEM((CHUNK // 128, 128), jnp.int32),
            pltpu.VMEM((CHUNK, HALF), jnp.float32),
        ],
    )
    def k(x_hbm, dst_hbm, z_hbm, s_hbm, acc_sh, idx_v, buf_v):
        c = lax.axis_index("c")
        s = lax.axis_index("s")
        col0 = c * HALF

        @pl.loop(0, -(-(N // ROWBLK) // SC_SUBCORES))
        def _(j):
            rb = j * SC_SUBCORES + s

            @pl.when(rb < N // ROWBLK)
            def _():
                pltpu.sync_copy(z_hbm, acc_sh.at[pl.ds(rb * ROWBLK, ROWBLK)])

        plsc.subcore_barrier()

        @pl.loop(0, nloop)
        def _(j):
            ci = j * SC_SUBCORES + s

            @pl.when(ci < nch)
            def _():
                base = ci * CHUNK
                pltpu.sync_copy(dst_hbm.at[pl.ds(ci * (CHUNK // 128), CHUNK // 128)], idx_v)
                pltpu.sync_copy(x_hbm.at[pl.ds(base, CHUNK), pl.ds(col0, HALF)], buf_v)
                for q in range(CHUNK // 128):
                    pltpu.sync_copy(buf_v.at[pl.ds(q * 128, 128)],
                                    acc_sh.at[idx_v.at[q]], add=True)

        plsc.subcore_barrier()

        @pl.loop(0, -(-(N // ROWBLK) // SC_SUBCORES))
        def _(j):
            rb = j * SC_SUBCORES + s

            @pl.when(rb < N // ROWBLK)
            def _():
                pltpu.sync_copy(acc_sh.at[pl.ds(rb * ROWBLK, ROWBLK)],
                                s_hbm.at[pl.ds(rb * ROWBLK, ROWBLK),
                                         pl.ds(col0, HALF)])

    return k(x, dst2, zrows)


def kernel(V, edge_attr, edge_index, rev_index, batch_ids,
           W_i, b_i, W_h, W_o, b_o, bn_gamma, bn_beta, bn_mean, bn_var,
           W_ffn, b_ffn):
    src = edge_index[0]
    dst = edge_index[1]
    W_i1 = W_i[:DV]
    W_i2 = W_i[DV:]
    W_o1 = W_o[:DV]
    W_o2 = W_o[DV:]
    b_i2 = b_i.reshape(1, HID)
    b_o2 = b_o.reshape(1, HID)
    zrows = jnp.zeros((ROWBLK, HALF), jnp.float32)
    ids3 = batch_ids.reshape(NNB, 1, NBLK)
    src2 = src.reshape(E // 128, 128)
    dst2 = dst.reshape(E // 128, 128)

    VWi, VWo = _node_matmuls(V, W_i1, W_o1, b_o2)
    G0 = _gather_rows(VWi, src2)
    h0, hW = _h0_and_hw(G0, edge_attr, W_i2, b_i2, W_h)
    G = _segsum_gather(hW, dst2, src2, zrows)
    hW = _step_hw(h0, G, hW, W_h)
    G = _segsum_gather(hW, dst2, src2, zrows)
    h3 = _step_h(h0, G, hW)
    m_v = _segsum(h3, dst2, zrows)
    return _final(VWo, m_v, W_o2, ids3, bn_gamma.reshape(1, HID),
                  bn_beta.reshape(1, HID), bn_mean.reshape(1, HID),
                  bn_var.reshape(1, HID), W_ffn, b_ffn.reshape(1, OUT))


# speedup vs baseline: 2.8800x; 1.1931x over previous
import functools

import jax
import jax.numpy as jnp
from jax import lax
from jax.experimental import pallas as pl
from jax.experimental.pallas import tpu as pltpu
from jax.experimental.pallas import tpu_sc as plsc

N = 10000
EH = 80000
E = 2 * EH
DV = 256
DE = 16
HID = 256
B = 64
OUT = 300

EBLK = 1000
NEB = E // EBLK
NBLK = 1000
NNB = N // NBLK

SC_CORES = 2
SC_SUBCORES = 16
CHUNK = 128
GCHUNK = 128
ROWBLK = 200
HALF = HID // 2


def _mm2_body(v_ref, wi1_ref, wo1_ref, bo_ref, vwi_ref, vwo_ref):
    v = v_ref[...]
    vwi_ref[...] = jnp.dot(v, wi1_ref[...], preferred_element_type=jnp.float32)
    vwo_ref[...] = jnp.dot(v, wo1_ref[...], preferred_element_type=jnp.float32) + bo_ref[...]


def _node_matmuls(V, W_i1, W_o1, b_o):
    return pl.pallas_call(
        _mm2_body,
        grid=(NNB,),
        in_specs=[
            pl.BlockSpec((NBLK, DV), lambda i: (i, 0)),
            pl.BlockSpec((DV, HID), lambda i: (0, 0)),
            pl.BlockSpec((DV, HID), lambda i: (0, 0)),
            pl.BlockSpec((1, HID), lambda i: (0, 0)),
        ],
        out_specs=[
            pl.BlockSpec((NBLK, HID), lambda i: (i, 0)),
            pl.BlockSpec((NBLK, HID), lambda i: (i, 0)),
        ],
        out_shape=[
            jax.ShapeDtypeStruct((N, HID), jnp.float32),
            jax.ShapeDtypeStruct((N, HID), jnp.float32),
        ],
        compiler_params=pltpu.CompilerParams(dimension_semantics=("parallel",)),
    )(V, W_i1, W_o1, b_o)


def _h0_hw_body(g_ref, ea_ref, wi2_ref, bi_ref, wh_ref, h_ref, hw_ref):
    x = g_ref[...] + jnp.dot(ea_ref[...], wi2_ref[...],
                             preferred_element_type=jnp.float32) + bi_ref[...]
    h = jnp.maximum(x, 0.0)
    h_ref[...] = h
    hw_ref[...] = jnp.dot(h, wh_ref[...], preferred_element_type=jnp.float32)


def _h0_and_hw(G0, edge_attr, W_i2, b_i, W_h):
    return pl.pallas_call(
        _h0_hw_body,
        grid=(NEB,),
        in_specs=[
            pl.BlockSpec((EBLK, HID), lambda i: (i, 0)),
            pl.BlockSpec((EBLK, DE), lambda i: (i, 0)),
            pl.BlockSpec((DE, HID), lambda i: (0, 0)),
            pl.BlockSpec((1, HID), lambda i: (0, 0)),
            pl.BlockSpec((HID, HID), lambda i: (0, 0)),
        ],
        out_specs=[
            pl.BlockSpec((EBLK, HID), lambda i: (i, 0)),
            pl.BlockSpec((EBLK, HID), lambda i: (i, 0)),
        ],
        out_shape=[
            jax.ShapeDtypeStruct((E, HID), jnp.float32),
            jax.ShapeDtypeStruct((E, HID), jnp.float32),
        ],
        compiler_params=pltpu.CompilerParams(dimension_semantics=("parallel",)),
    )(G0, edge_attr, W_i2, b_i, W_h)


def _step_hw_body(h0_ref, g_ref, hwp_ref, wh_ref, hw_ref):
    h = jnp.maximum(h0_ref[...] + g_ref[...] - hwp_ref[...], 0.0)
    hw_ref[...] = jnp.dot(h, wh_ref[...], preferred_element_type=jnp.float32)


def _step_hw(h0, G, hWprev, W_h):
    return pl.pallas_call(
        _step_hw_body,
        grid=(NEB,),
        in_specs=[
            pl.BlockSpec((EBLK, HID), lambda i: (i, 0)),
            pl.BlockSpec((EBLK, HID), lambda i: (i, 0)),
            pl.BlockSpec((EBLK, HID), lambda i: ((i + NEB // 2) % NEB, 0)),
            pl.BlockSpec((HID, HID), lambda i: (0, 0)),
        ],
        out_specs=pl.BlockSpec((EBLK, HID), lambda i: (i, 0)),
        out_shape=jax.ShapeDtypeStruct((E, HID), jnp.float32),
        compiler_params=pltpu.CompilerParams(dimension_semantics=("parallel",)),
    )(h0, G, hWprev, W_h)


def _step_h_body(h0_ref, g_ref, hwp_ref, h_ref):
    h_ref[...] = jnp.maximum(h0_ref[...] + g_ref[...] - hwp_ref[...], 0.0)


def _step_h(h0, G, hWprev):
    return pl.pallas_call(
        _step_h_body,
        grid=(NEB,),
        in_specs=[
            pl.BlockSpec((EBLK, HID), lambda i: (i, 0)),
            pl.BlockSpec((EBLK, HID), lambda i: (i, 0)),
            pl.BlockSpec((EBLK, HID), lambda i: ((i + NEB // 2) % NEB, 0)),
        ],
        out_specs=pl.BlockSpec((EBLK, HID), lambda i: (i, 0)),
        out_shape=jax.ShapeDtypeStruct((E, HID), jnp.float32),
        compiler_params=pltpu.CompilerParams(dimension_semantics=("parallel",)),
    )(h0, G, hWprev)


def _final_body(vwo_ref, mv_ref, wo2_ref, ids_ref, gam_ref, bet_ref,
                mean_ref, var_ref, wf_ref, bf_ref, out_ref,
                sum_ref, cnt_ref):
    i = pl.program_id(0)
    hv = jnp.maximum(
        vwo_ref[...] + jnp.dot(mv_ref[...], wo2_ref[...],
                               preferred_element_type=jnp.float32), 0.0)
    ids = ids_ref[0, 0, :]
    oneh = (lax.broadcasted_iota(jnp.int32, (NBLK, B), 1)
            == ids[:, None]).astype(jnp.float32)
    psum = lax.dot_general(oneh, hv, (((0,), (0,)), ((), ())),
                           preferred_element_type=jnp.float32)
    pcnt = lax.dot_general(oneh, jnp.ones((NBLK, 128), jnp.float32),
                           (((0,), (0,)), ((), ())),
                           preferred_element_type=jnp.float32)

    @pl.when(i == 0)
    def _():
        sum_ref[...] = jnp.zeros_like(sum_ref)
        cnt_ref[...] = jnp.zeros_like(cnt_ref)

    sum_ref[...] += psum
    cnt_ref[...] += pcnt

    @pl.when(i == NNB - 1)
    def _():
        counts = jnp.maximum(cnt_ref[:, 0:1], 1.0)
        hm = sum_ref[...] / counts
        hn = (gam_ref[...] * (hm - mean_ref[...])
              * lax.rsqrt(var_ref[...] + 1e-5) + bet_ref[...])
        out_ref[...] = jnp.maximum(
            jnp.dot(hn, wf_ref[...], preferred_element_type=jnp.float32)
            + bf_ref[...], 0.0)


def _final(VWo, m_v, W_o2, ids3, bn_gamma, bn_beta, bn_mean, bn_var,
           W_ffn, b_ffn):
    return pl.pallas_call(
        _final_body,
        grid=(NNB,),
        in_specs=[
            pl.BlockSpec((NBLK, HID), lambda i: (i, 0)),
            pl.BlockSpec((NBLK, HID), lambda i: (i, 0)),
            pl.BlockSpec((HID, HID), lambda i: (0, 0)),
            pl.BlockSpec((1, 1, NBLK), lambda i: (i, 0, 0)),
            pl.BlockSpec((1, HID), lambda i: (0, 0)),
            pl.BlockSpec((1, HID), lambda i: (0, 0)),
            pl.BlockSpec((1, HID), lambda i: (0, 0)),
            pl.BlockSpec((1, HID), lambda i: (0, 0)),
            pl.BlockSpec((HID, OUT), lambda i: (0, 0)),
            pl.BlockSpec((1, OUT), lambda i: (0, 0)),
        ],
        out_specs=pl.BlockSpec((B, OUT), lambda i: (0, 0)),
        out_shape=jax.ShapeDtypeStruct((B, OUT), jnp.float32),
        scratch_shapes=[
            pltpu.VMEM((B, HID), jnp.float32),
            pltpu.VMEM((B, 128), jnp.float32),
        ],
    )(VWo, m_v, W_o2, ids3, bn_gamma, bn_beta, bn_mean, bn_var, W_ffn, b_ffn)


def _sc_mesh():
    return plsc.VectorSubcoreMesh(core_axis_name="c", subcore_axis_name="s")


def _gather_rows(table, src2):
    nch = E // GCHUNK
    nw = SC_CORES * SC_SUBCORES
    per = nch // nw
    nextra = nch - per * nw

    @functools.partial(
        pl.kernel,
        mesh=_sc_mesh(),
        out_type=jax.ShapeDtypeStruct((E, HID), jnp.float32),
        scratch_types=[
            pltpu.VMEM((per + 9, GCHUNK), jnp.int32),
            pltpu.VMEM((2, GCHUNK, HID), jnp.float32),
            pltpu.SemaphoreType.DMA((2,)),
            pltpu.SemaphoreType.DMA((2,)),
        ],
    )
    def k(table_hbm, src_hbm, out_hbm, idx_v, buf_v, gsem, wsem):
        c = lax.axis_index("c")
        s = lax.axis_index("s")
        wid = s * SC_CORES + c
        lo = wid * per
        lo8 = (lo // 8) * 8
        off = lo - lo8

        pltpu.sync_copy(src_hbm.at[pl.ds(lo8, per + 9)], idx_v)
        pltpu.async_copy(table_hbm.at[idx_v.at[off]], buf_v.at[0], gsem.at[0])

        def body(j, p):
            pltpu.make_async_copy(table_hbm.at[idx_v.at[off]], buf_v.at[p],
                                  gsem.at[p]).wait()

            @pl.when(j + 1 < per)
            def _():
                @pl.when(j >= 1)
                def _():
                    pltpu.make_async_copy(
                        buf_v.at[1 - p],
                        out_hbm.at[pl.ds(lo * GCHUNK, GCHUNK)],
                        wsem.at[1 - p]).wait()

                pltpu.async_copy(table_hbm.at[idx_v.at[off + j + 1]],
                                 buf_v.at[1 - p], gsem.at[1 - p])

            pltpu.async_copy(buf_v.at[p],
                             out_hbm.at[pl.ds((lo + j) * GCHUNK, GCHUNK)],
                             wsem.at[p])

        @pl.loop(0, (per + 1) // 2)
        def _(jj):
            for p in range(2):
                @pl.when(jj * 2 + p < per)
                def _(j=jj * 2 + p, p=p):
                    body(j, p)

        pltpu.make_async_copy(buf_v.at[(per - 1) % 2],
                              out_hbm.at[pl.ds(lo * GCHUNK, GCHUNK)],
                              wsem.at[(per - 1) % 2]).wait()
        pltpu.make_async_copy(buf_v.at[per % 2],
                              out_hbm.at[pl.ds(lo * GCHUNK, GCHUNK)],
                              wsem.at[per % 2]).wait()

        @pl.when(wid < nextra)
        def _():
            ci = nch - nextra + wid
            pltpu.sync_copy(src_hbm.at[ci], idx_v.at[0])
            pltpu.sync_copy(table_hbm.at[idx_v.at[0]], buf_v.at[0])
            pltpu.sync_copy(buf_v.at[0], out_hbm.at[pl.ds(ci * GCHUNK, GCHUNK)])

    return k(table, src2)


def _sc_zero_acc(z_hbm, acc_sh, s):
    @pl.loop(0, -(-(N // ROWBLK) // SC_SUBCORES))
    def _(j):
        rb = j * SC_SUBCORES + s

        @pl.when(rb < N // ROWBLK)
        def _():
            pltpu.sync_copy(z_hbm, acc_sh.at[pl.ds(rb * ROWBLK, ROWBLK)])


def _sc_scatter_phase(x_hbm, dst_hbm, acc_sh, idx_v, buf_v, dsem, s, col0):
    nch = E // CHUNK
    per = nch // SC_SUBCORES
    nextra = nch - per * SC_SUBCORES
    lo = s * per
    row8 = (lo // 8) * 8
    off = lo - row8

    pltpu.sync_copy(dst_hbm.at[pl.ds(row8, per + 10)], idx_v)
    pltpu.async_copy(x_hbm.at[pl.ds(lo * CHUNK, CHUNK), pl.ds(col0, HALF)],
                     buf_v.at[0], dsem.at[0])

    def body(j, p):
        pltpu.make_async_copy(
            x_hbm.at[pl.ds(lo * CHUNK, CHUNK), pl.ds(col0, HALF)],
            buf_v.at[p], dsem.at[p]).wait()

        @pl.when(j + 1 < per)
        def _():
            pltpu.async_copy(
                x_hbm.at[pl.ds((lo + j + 1) * CHUNK, CHUNK), pl.ds(col0, HALF)],
                buf_v.at[1 - p], dsem.at[1 - p])

        pltpu.sync_copy(buf_v.at[p], acc_sh.at[idx_v.at[off + j]], add=True)

    @pl.loop(0, (per + 1) // 2)
    def _(jj):
        for p in range(2):
            @pl.when(jj * 2 + p < per)
            def _(j=jj * 2 + p, p=p):
                body(j, p)

    @pl.when(s < nextra)
    def _():
        ci = nch - nextra + s
        pltpu.sync_copy(dst_hbm.at[ci], idx_v.at[0])
        pltpu.sync_copy(x_hbm.at[pl.ds(ci * CHUNK, CHUNK), pl.ds(col0, HALF)],
                        buf_v.at[0])
        pltpu.sync_copy(buf_v.at[0], acc_sh.at[idx_v.at[0]], add=True)


def _segsum_gather(x, dst2, src2, zrows):
    nch = E // CHUNK
    per = nch // SC_SUBCORES

    @functools.partial(
        pl.kernel,
        mesh=_sc_mesh(),
        out_type=jax.ShapeDtypeStruct((E, HID), jnp.float32),
        scratch_types=[
            pltpu.VMEM_SHARED((N, HALF), jnp.float32),
            pltpu.VMEM((per + 10, 128), jnp.int32),
            pltpu.VMEM((2, CHUNK, HALF), jnp.float32),
            pltpu.SemaphoreType.DMA((2,)),
            pltpu.SemaphoreType.DMA((2,)),
        ],
    )
    def k(x_hbm, dst_hbm, src_hbm, z_hbm, g_hbm, acc_sh, idx_v, buf_v,
          dsem, wsem):
        c = lax.axis_index("c")
        s = lax.axis_index("s")
        col0 = c * HALF
        lo = s * per

        _sc_zero_acc(z_hbm, acc_sh, s)
        plsc.subcore_barrier()
        _sc_scatter_phase(x_hbm, dst_hbm, acc_sh, idx_v, buf_v, dsem, s, col0)
        plsc.subcore_barrier()

        row8 = (lo // 8) * 8
        off = lo - row8
        pltpu.sync_copy(src_hbm.at[pl.ds(row8, per + 10)], idx_v)

        def gbody(j, p):
            @pl.when(j >= 2)
            def _():
                pltpu.make_async_copy(
                    buf_v.at[p],
                    g_hbm.at[pl.ds(lo * CHUNK, CHUNK), pl.ds(col0, HALF)],
                    wsem.at[p]).wait()

            pltpu.sync_copy(acc_sh.at[idx_v.at[off + j]], buf_v.at[p])
            pltpu.async_copy(
                buf_v.at[p],
                g_hbm.at[pl.ds((lo + j) * CHUNK, CHUNK), pl.ds(col0, HALF)],
                wsem.at[p])

        @pl.loop(0, (per + 1) // 2)
        def _(jj):
            for p in range(2):
                @pl.when(jj * 2 + p < per)
                def _(j=jj * 2 + p, p=p):
                    gbody(j, p)

        pltpu.make_async_copy(
            buf_v.at[(per - 1) % 2],
            g_hbm.at[pl.ds(lo * CHUNK, CHUNK), pl.ds(col0, HALF)],
            wsem.at[(per - 1) % 2]).wait()
        pltpu.make_async_copy(
            buf_v.at[per % 2],
            g_hbm.at[pl.ds(lo * CHUNK, CHUNK), pl.ds(col0, HALF)],
            wsem.at[per % 2]).wait()

        @pl.when(s < nch - per * SC_SUBCORES)
        def _():
            ci = per * SC_SUBCORES + s
            pltpu.sync_copy(src_hbm.at[ci], idx_v.at[0])
            pltpu.sync_copy(acc_sh.at[idx_v.at[0]], buf_v.at[0])
            pltpu.sync_copy(buf_v.at[0],
                            g_hbm.at[pl.ds(ci * CHUNK, CHUNK),
                                     pl.ds(col0, HALF)])

    return k(x, dst2, src2, zrows)


def _segsum(x, dst2, zrows):
    nch = E // CHUNK
    per = nch // SC_SUBCORES

    @functools.partial(
        pl.kernel,
        mesh=_sc_mesh(),
        out_type=jax.ShapeDtypeStruct((N, HID), jnp.float32),
        scratch_types=[
            pltpu.VMEM_SHARED((N, HALF), jnp.float32),
            pltpu.VMEM((per + 10, 128), jnp.int32),
            pltpu.VMEM((2, CHUNK, HALF), jnp.float32),
            pltpu.SemaphoreType.DMA((2,)),
        ],
    )
    def k(x_hbm, dst_hbm, z_hbm, s_hbm, acc_sh, idx_v, buf_v, dsem):
        c = lax.axis_index("c")
        s = lax.axis_index("s")
        col0 = c * HALF

        _sc_zero_acc(z_hbm, acc_sh, s)
        plsc.subcore_barrier()
        _sc_scatter_phase(x_hbm, dst_hbm, acc_sh, idx_v, buf_v, dsem, s, col0)
        plsc.subcore_barrier()

        @pl.loop(0, -(-(N // ROWBLK) // SC_SUBCORES))
        def _(j):
            rb = j * SC_SUBCORES + s

            @pl.when(rb < N // ROWBLK)
            def _():
                pltpu.sync_copy(acc_sh.at[pl.ds(rb * ROWBLK, ROWBLK)],
                                s_hbm.at[pl.ds(rb * ROWBLK, ROWBLK),
                                         pl.ds(col0, HALF)])

    return k(x, dst2, zrows)


def kernel(V, edge_attr, edge_index, rev_index, batch_ids,
           W_i, b_i, W_h, W_o, b_o, bn_gamma, bn_beta, bn_mean, bn_var,
           W_ffn, b_ffn):
    src = edge_index[0]
    dst = edge_index[1]
    W_i1 = W_i[:DV]
    W_i2 = W_i[DV:]
    W_o1 = W_o[:DV]
    W_o2 = W_o[DV:]
    b_i2 = b_i.reshape(1, HID)
    b_o2 = b_o.reshape(1, HID)
    zrows = jnp.zeros((ROWBLK, HALF), jnp.float32)
    ids3 = batch_ids.reshape(NNB, 1, NBLK)
    src2 = jnp.pad(src.reshape(E // 128, 128), ((0, 6), (0, 0)))
    dst2 = jnp.pad(dst.reshape(E // 128, 128), ((0, 6), (0, 0)))

    VWi, VWo = _node_matmuls(V, W_i1, W_o1, b_o2)
    G0 = _gather_rows(VWi, src2)
    h0, hW = _h0_and_hw(G0, edge_attr, W_i2, b_i2, W_h)
    G = _segsum_gather(hW, dst2, src2, zrows)
    hW = _step_hw(h0, G, hW, W_h)
    G = _segsum_gather(hW, dst2, src2, zrows)
    h3 = _step_h(h0, G, hW)
    m_v = _segsum(h3, dst2, zrows)
    return _final(VWo, m_v, W_o2, ids3, bn_gamma.reshape(1, HID),
                  bn_beta.reshape(1, HID), bn_mean.reshape(1, HID),
                  bn_var.reshape(1, HID), W_ffn, b_ffn.reshape(1, OUT))


# speedup vs baseline: 2.9802x; 1.0348x over previous
import functools

import jax
import jax.numpy as jnp
from jax import lax
from jax.experimental import pallas as pl
from jax.experimental.pallas import tpu as pltpu
from jax.experimental.pallas import tpu_sc as plsc

N = 10000
EH = 80000
E = 2 * EH
DV = 256
DE = 16
HID = 256
B = 64
OUT = 300

EBLK = 1000
NEB = E // EBLK
NBLK = 1000
NNB = N // NBLK

SC_CORES = 2
SC_SUBCORES = 16
CHUNK = 128
GCHUNK = 128
ROWBLK = 200
HALF = HID // 2


def _mm2_body(v_ref, wi1_ref, wo1_ref, bo_ref, vwi_ref, vwo_ref):
    v = v_ref[...]
    vwi_ref[...] = jnp.dot(v, wi1_ref[...], preferred_element_type=jnp.float32)
    vwo_ref[...] = jnp.dot(v, wo1_ref[...], preferred_element_type=jnp.float32) + bo_ref[...]


def _node_matmuls(V, W_i1, W_o1, b_o):
    return pl.pallas_call(
        _mm2_body,
        grid=(NNB,),
        in_specs=[
            pl.BlockSpec((NBLK, DV), lambda i: (i, 0)),
            pl.BlockSpec((DV, HID), lambda i: (0, 0)),
            pl.BlockSpec((DV, HID), lambda i: (0, 0)),
            pl.BlockSpec((1, HID), lambda i: (0, 0)),
        ],
        out_specs=[
            pl.BlockSpec((NBLK, HID), lambda i: (i, 0)),
            pl.BlockSpec((NBLK, HID), lambda i: (i, 0)),
        ],
        out_shape=[
            jax.ShapeDtypeStruct((N, HID), jnp.float32),
            jax.ShapeDtypeStruct((N, HID), jnp.float32),
        ],
        compiler_params=pltpu.CompilerParams(dimension_semantics=("parallel",)),
    )(V, W_i1, W_o1, b_o)


def _h0_hw_body(g_ref, ea_ref, wi2_ref, bi_ref, wh_ref, h_ref, hw_ref):
    x = g_ref[...] + jnp.dot(ea_ref[...], wi2_ref[...],
                             preferred_element_type=jnp.float32) + bi_ref[...]
    h = jnp.maximum(x, 0.0)
    hb = h.astype(jnp.bfloat16)
    h_ref[...] = hb
    hw_ref[...] = jnp.dot(hb, wh_ref[...], preferred_element_type=jnp.float32)


def _h0_and_hw(G0, edge_attr, W_i2, b_i, W_h):
    return pl.pallas_call(
        _h0_hw_body,
        grid=(NEB,),
        in_specs=[
            pl.BlockSpec((EBLK, HID), lambda i: (i, 0)),
            pl.BlockSpec((EBLK, DE), lambda i: (i, 0)),
            pl.BlockSpec((DE, HID), lambda i: (0, 0)),
            pl.BlockSpec((1, HID), lambda i: (0, 0)),
            pl.BlockSpec((HID, HID), lambda i: (0, 0)),
        ],
        out_specs=[
            pl.BlockSpec((EBLK, HID), lambda i: (i, 0)),
            pl.BlockSpec((EBLK, HID), lambda i: (i, 0)),
        ],
        out_shape=[
            jax.ShapeDtypeStruct((E, HID), jnp.bfloat16),
            jax.ShapeDtypeStruct((E, HID), jnp.float32),
        ],
        compiler_params=pltpu.CompilerParams(dimension_semantics=("parallel",)),
    )(G0, edge_attr, W_i2, b_i, W_h)


def _step_hw_body(h0_ref, g_ref, hwp_ref, wh_ref, hw_ref):
    h = jnp.maximum(h0_ref[...].astype(jnp.float32) + g_ref[...] - hwp_ref[...],
                    0.0)
    hw_ref[...] = jnp.dot(h.astype(jnp.bfloat16), wh_ref[...],
                          preferred_element_type=jnp.float32)


def _step_hw(h0, G, hWprev, W_h):
    return pl.pallas_call(
        _step_hw_body,
        grid=(NEB,),
        in_specs=[
            pl.BlockSpec((EBLK, HID), lambda i: (i, 0)),
            pl.BlockSpec((EBLK, HID), lambda i: (i, 0)),
            pl.BlockSpec((EBLK, HID), lambda i: ((i + NEB // 2) % NEB, 0)),
            pl.BlockSpec((HID, HID), lambda i: (0, 0)),
        ],
        out_specs=pl.BlockSpec((EBLK, HID), lambda i: (i, 0)),
        out_shape=jax.ShapeDtypeStruct((E, HID), jnp.float32),
        compiler_params=pltpu.CompilerParams(dimension_semantics=("parallel",)),
    )(h0, G, hWprev, W_h)


def _step_h_body(h0_ref, g_ref, hwp_ref, h_ref):
    h_ref[...] = jnp.maximum(
        h0_ref[...].astype(jnp.float32) + g_ref[...] - hwp_ref[...], 0.0)


def _step_h(h0, G, hWprev):
    return pl.pallas_call(
        _step_h_body,
        grid=(NEB,),
        in_specs=[
            pl.BlockSpec((EBLK, HID), lambda i: (i, 0)),
            pl.BlockSpec((EBLK, HID), lambda i: (i, 0)),
            pl.BlockSpec((EBLK, HID), lambda i: ((i + NEB // 2) % NEB, 0)),
        ],
        out_specs=pl.BlockSpec((EBLK, HID), lambda i: (i, 0)),
        out_shape=jax.ShapeDtypeStruct((E, HID), jnp.float32),
        compiler_params=pltpu.CompilerParams(dimension_semantics=("parallel",)),
    )(h0, G, hWprev)


def _final_body(vwo_ref, mv_ref, wo2_ref, ids_ref, gam_ref, bet_ref,
                mean_ref, var_ref, wf_ref, bf_ref, out_ref,
                sum_ref, cnt_ref):
    i = pl.program_id(0)
    hv = jnp.maximum(
        vwo_ref[...] + jnp.dot(mv_ref[...], wo2_ref[...],
                               preferred_element_type=jnp.float32), 0.0)
    ids = ids_ref[0, 0, :]
    oneh = (lax.broadcasted_iota(jnp.int32, (NBLK, B), 1)
            == ids[:, None]).astype(jnp.float32)
    psum = lax.dot_general(oneh, hv, (((0,), (0,)), ((), ())),
                           preferred_element_type=jnp.float32)
    pcnt = lax.dot_general(oneh, jnp.ones((NBLK, 128), jnp.float32),
                           (((0,), (0,)), ((), ())),
                           preferred_element_type=jnp.float32)

    @pl.when(i == 0)
    def _():
        sum_ref[...] = jnp.zeros_like(sum_ref)
        cnt_ref[...] = jnp.zeros_like(cnt_ref)

    sum_ref[...] += psum
    cnt_ref[...] += pcnt

    @pl.when(i == NNB - 1)
    def _():
        counts = jnp.maximum(cnt_ref[:, 0:1], 1.0)
        hm = sum_ref[...] / counts
        hn = (gam_ref[...] * (hm - mean_ref[...])
              * lax.rsqrt(var_ref[...] + 1e-5) + bet_ref[...])
        out_ref[...] = jnp.maximum(
            jnp.dot(hn, wf_ref[...], preferred_element_type=jnp.float32)
            + bf_ref[...], 0.0)


def _final(VWo, m_v, W_o2, ids3, bn_gamma, bn_beta, bn_mean, bn_var,
           W_ffn, b_ffn):
    return pl.pallas_call(
        _final_body,
        grid=(NNB,),
        in_specs=[
            pl.BlockSpec((NBLK, HID), lambda i: (i, 0)),
            pl.BlockSpec((NBLK, HID), lambda i: (i, 0)),
            pl.BlockSpec((HID, HID), lambda i: (0, 0)),
            pl.BlockSpec((1, 1, NBLK), lambda i: (i, 0, 0)),
            pl.BlockSpec((1, HID), lambda i: (0, 0)),
            pl.BlockSpec((1, HID), lambda i: (0, 0)),
            pl.BlockSpec((1, HID), lambda i: (0, 0)),
            pl.BlockSpec((1, HID), lambda i: (0, 0)),
            pl.BlockSpec((HID, OUT), lambda i: (0, 0)),
            pl.BlockSpec((1, OUT), lambda i: (0, 0)),
        ],
        out_specs=pl.BlockSpec((B, OUT), lambda i: (0, 0)),
        out_shape=jax.ShapeDtypeStruct((B, OUT), jnp.float32),
        scratch_shapes=[
            pltpu.VMEM((B, HID), jnp.float32),
            pltpu.VMEM((B, 128), jnp.float32),
        ],
    )(VWo, m_v, W_o2, ids3, bn_gamma, bn_beta, bn_mean, bn_var, W_ffn, b_ffn)


def _sc_mesh():
    return plsc.VectorSubcoreMesh(core_axis_name="c", subcore_axis_name="s")


def _gather_rows(table, src2):
    nch = E // GCHUNK
    nw = SC_CORES * SC_SUBCORES
    per = nch // nw
    nextra = nch - per * nw

    @functools.partial(
        pl.kernel,
        mesh=_sc_mesh(),
        out_type=jax.ShapeDtypeStruct((E, HID), jnp.float32),
        scratch_types=[
            pltpu.VMEM((per + 9, GCHUNK), jnp.int32),
            pltpu.VMEM((2, GCHUNK, HID), jnp.float32),
            pltpu.SemaphoreType.DMA((2,)),
            pltpu.SemaphoreType.DMA((2,)),
        ],
    )
    def k(table_hbm, src_hbm, out_hbm, idx_v, buf_v, gsem, wsem):
        c = lax.axis_index("c")
        s = lax.axis_index("s")
        wid = s * SC_CORES + c
        lo = wid * per
        lo8 = (lo // 8) * 8
        off = lo - lo8

        pltpu.sync_copy(src_hbm.at[pl.ds(lo8, per + 9)], idx_v)
        pltpu.async_copy(table_hbm.at[idx_v.at[off]], buf_v.at[0], gsem.at[0])

        def body(j, p):
            pltpu.make_async_copy(table_hbm.at[idx_v.at[off]], buf_v.at[p],
                                  gsem.at[p]).wait()

            @pl.when(j + 1 < per)
            def _():
                @pl.when(j >= 1)
                def _():
                    pltpu.make_async_copy(
                        buf_v.at[1 - p],
                        out_hbm.at[pl.ds(lo * GCHUNK, GCHUNK)],
                        wsem.at[1 - p]).wait()

                pltpu.async_copy(table_hbm.at[idx_v.at[off + j + 1]],
                                 buf_v.at[1 - p], gsem.at[1 - p])

            pltpu.async_copy(buf_v.at[p],
                             out_hbm.at[pl.ds((lo + j) * GCHUNK, GCHUNK)],
                             wsem.at[p])

        @pl.loop(0, (per + 1) // 2)
        def _(jj):
            for p in range(2):
                @pl.when(jj * 2 + p < per)
                def _(j=jj * 2 + p, p=p):
                    body(j, p)

        pltpu.make_async_copy(buf_v.at[(per - 1) % 2],
                              out_hbm.at[pl.ds(lo * GCHUNK, GCHUNK)],
                              wsem.at[(per - 1) % 2]).wait()
        pltpu.make_async_copy(buf_v.at[per % 2],
                              out_hbm.at[pl.ds(lo * GCHUNK, GCHUNK)],
                              wsem.at[per % 2]).wait()

        @pl.when(wid < nextra)
        def _():
            ci = nch - nextra + wid
            pltpu.sync_copy(src_hbm.at[ci], idx_v.at[0])
            pltpu.sync_copy(table_hbm.at[idx_v.at[0]], buf_v.at[0])
            pltpu.sync_copy(buf_v.at[0], out_hbm.at[pl.ds(ci * GCHUNK, GCHUNK)])

    return k(table, src2)


def _sc_zero_acc(z_hbm, acc_sh, s):
    @pl.loop(0, -(-(N // ROWBLK) // SC_SUBCORES))
    def _(j):
        rb = j * SC_SUBCORES + s

        @pl.when(rb < N // ROWBLK)
        def _():
            pltpu.sync_copy(z_hbm, acc_sh.at[pl.ds(rb * ROWBLK, ROWBLK)])


def _sc_scatter_phase(x_hbm, dst_hbm, acc_sh, idx_v, buf_v, dsem, s, col0):
    nch = E // CHUNK
    per = nch // SC_SUBCORES
    nextra = nch - per * SC_SUBCORES
    lo = s * per
    row8 = (lo // 8) * 8
    off = lo - row8

    pltpu.sync_copy(dst_hbm.at[pl.ds(row8, per + 10)], idx_v)
    pltpu.async_copy(x_hbm.at[pl.ds(lo * CHUNK, CHUNK), pl.ds(col0, HALF)],
                     buf_v.at[0], dsem.at[0])

    def body(j, p):
        pltpu.make_async_copy(
            x_hbm.at[pl.ds(lo * CHUNK, CHUNK), pl.ds(col0, HALF)],
            buf_v.at[p], dsem.at[p]).wait()

        @pl.when(j + 1 < per)
        def _():
            pltpu.async_copy(
                x_hbm.at[pl.ds((lo + j + 1) * CHUNK, CHUNK), pl.ds(col0, HALF)],
                buf_v.at[1 - p], dsem.at[1 - p])

        pltpu.sync_copy(buf_v.at[p], acc_sh.at[idx_v.at[off + j]], add=True)

    @pl.loop(0, (per + 1) // 2)
    def _(jj):
        for p in range(2):
            @pl.when(jj * 2 + p < per)
            def _(j=jj * 2 + p, p=p):
                body(j, p)

    @pl.when(s < nextra)
    def _():
        ci = nch - nextra + s
        pltpu.sync_copy(dst_hbm.at[ci], idx_v.at[0])
        pltpu.sync_copy(x_hbm.at[pl.ds(ci * CHUNK, CHUNK), pl.ds(col0, HALF)],
                        buf_v.at[0])
        pltpu.sync_copy(buf_v.at[0], acc_sh.at[idx_v.at[0]], add=True)


def _segsum_gather(x, dst2, src2, zrows):
    nch = E // CHUNK
    per = nch // SC_SUBCORES

    @functools.partial(
        pl.kernel,
        mesh=_sc_mesh(),
        out_type=jax.ShapeDtypeStruct((E, HID), jnp.float32),
        scratch_types=[
            pltpu.VMEM_SHARED((N, HALF), jnp.float32),
            pltpu.VMEM((per + 10, 128), jnp.int32),
            pltpu.VMEM((2, CHUNK, HALF), jnp.float32),
            pltpu.SemaphoreType.DMA((2,)),
            pltpu.SemaphoreType.DMA((2,)),
        ],
    )
    def k(x_hbm, dst_hbm, src_hbm, z_hbm, g_hbm, acc_sh, idx_v, buf_v,
          dsem, wsem):
        c = lax.axis_index("c")
        s = lax.axis_index("s")
        col0 = c * HALF
        lo = s * per

        _sc_zero_acc(z_hbm, acc_sh, s)
        plsc.subcore_barrier()
        _sc_scatter_phase(x_hbm, dst_hbm, acc_sh, idx_v, buf_v, dsem, s, col0)
        plsc.subcore_barrier()

        row8 = (lo // 8) * 8
        off = lo - row8
        pltpu.sync_copy(src_hbm.at[pl.ds(row8, per + 10)], idx_v)

        def gbody(j, p):
            @pl.when(j >= 2)
            def _():
                pltpu.make_async_copy(
                    buf_v.at[p],
                    g_hbm.at[pl.ds(lo * CHUNK, CHUNK), pl.ds(col0, HALF)],
                    wsem.at[p]).wait()

            pltpu.sync_copy(acc_sh.at[idx_v.at[off + j]], buf_v.at[p])
            pltpu.async_copy(
                buf_v.at[p],
                g_hbm.at[pl.ds((lo + j) * CHUNK, CHUNK), pl.ds(col0, HALF)],
                wsem.at[p])

        @pl.loop(0, (per + 1) // 2)
        def _(jj):
            for p in range(2):
                @pl.when(jj * 2 + p < per)
                def _(j=jj * 2 + p, p=p):
                    gbody(j, p)

        pltpu.make_async_copy(
            buf_v.at[(per - 1) % 2],
            g_hbm.at[pl.ds(lo * CHUNK, CHUNK), pl.ds(col0, HALF)],
            wsem.at[(per - 1) % 2]).wait()
        pltpu.make_async_copy(
            buf_v.at[per % 2],
            g_hbm.at[pl.ds(lo * CHUNK, CHUNK), pl.ds(col0, HALF)],
            wsem.at[per % 2]).wait()

        @pl.when(s < nch - per * SC_SUBCORES)
        def _():
            ci = per * SC_SUBCORES + s
            pltpu.sync_copy(src_hbm.at[ci], idx_v.at[0])
            pltpu.sync_copy(acc_sh.at[idx_v.at[0]], buf_v.at[0])
            pltpu.sync_copy(buf_v.at[0],
                            g_hbm.at[pl.ds(ci * CHUNK, CHUNK),
                                     pl.ds(col0, HALF)])

    return k(x, dst2, src2, zrows)


def _segsum(x, dst2, zrows):
    nch = E // CHUNK
    per = nch // SC_SUBCORES

    @functools.partial(
        pl.kernel,
        mesh=_sc_mesh(),
        out_type=jax.ShapeDtypeStruct((N, HID), jnp.float32),
        scratch_types=[
            pltpu.VMEM_SHARED((N, HALF), jnp.float32),
            pltpu.VMEM((per + 10, 128), jnp.int32),
            pltpu.VMEM((2, CHUNK, HALF), jnp.float32),
            pltpu.SemaphoreType.DMA((2,)),
        ],
    )
    def k(x_hbm, dst_hbm, z_hbm, s_hbm, acc_sh, idx_v, buf_v, dsem):
        c = lax.axis_index("c")
        s = lax.axis_index("s")
        col0 = c * HALF

        _sc_zero_acc(z_hbm, acc_sh, s)
        plsc.subcore_barrier()
        _sc_scatter_phase(x_hbm, dst_hbm, acc_sh, idx_v, buf_v, dsem, s, col0)
        plsc.subcore_barrier()

        @pl.loop(0, -(-(N // ROWBLK) // SC_SUBCORES))
        def _(j):
            rb = j * SC_SUBCORES + s

            @pl.when(rb < N // ROWBLK)
            def _():
                pltpu.sync_copy(acc_sh.at[pl.ds(rb * ROWBLK, ROWBLK)],
                                s_hbm.at[pl.ds(rb * ROWBLK, ROWBLK),
                                         pl.ds(col0, HALF)])

    return k(x, dst2, zrows)


def kernel(V, edge_attr, edge_index, rev_index, batch_ids,
           W_i, b_i, W_h, W_o, b_o, bn_gamma, bn_beta, bn_mean, bn_var,
           W_ffn, b_ffn):
    src = edge_index[0]
    dst = edge_index[1]
    W_i1 = W_i[:DV]
    W_i2 = W_i[DV:]
    W_o1 = W_o[:DV]
    W_o2 = W_o[DV:]
    b_i2 = b_i.reshape(1, HID)
    b_o2 = b_o.reshape(1, HID)
    zrows = jnp.zeros((ROWBLK, HALF), jnp.float32)
    ids3 = batch_ids.reshape(NNB, 1, NBLK)
    src2 = jnp.pad(src.reshape(E // 128, 128), ((0, 6), (0, 0)))
    dst2 = jnp.pad(dst.reshape(E // 128, 128), ((0, 6), (0, 0)))

    W_h_bf = W_h.astype(jnp.bfloat16)
    VWi, VWo = _node_matmuls(V, W_i1, W_o1, b_o2)
    G0 = _gather_rows(VWi, src2)
    h0, hW = _h0_and_hw(G0, edge_attr, W_i2, b_i2, W_h_bf)
    G = _segsum_gather(hW, dst2, src2, zrows)
    hW = _step_hw(h0, G, hW, W_h_bf)
    G = _segsum_gather(hW, dst2, src2, zrows)
    h3 = _step_h(h0, G, hW)
    m_v = _segsum(h3, dst2, zrows)
    return _final(VWo, m_v, W_o2, ids3, bn_gamma.reshape(1, HID),
                  bn_beta.reshape(1, HID), bn_mean.reshape(1, HID),
                  bn_var.reshape(1, HID), W_ffn, b_ffn.reshape(1, OUT))
